# baseline (speedup 1.0000x reference)
import functools

import jax
import jax.numpy as jnp
from jax import lax
from jax.experimental import pallas as pl
from jax.experimental.pallas import tpu as pltpu
from jax.experimental.pallas import tpu_sc as plsc

B = 4096
S = 200
D = 32
V = 1000000
NC = 2
NS = 16
NW = NC * NS
BBLK = B // NW
SBLK = 8
NBLK = S // SBLK
LANES = 16


@functools.partial(
    pl.kernel,
    out_type=jax.ShapeDtypeStruct((S, D // 8, NW, 8, BBLK), jnp.float32),
    mesh=plsc.VectorSubcoreMesh(core_axis_name="c", subcore_axis_name="s"),
    scratch_types=[
        pltpu.VMEM((2, 1, 1, SBLK, BBLK), jnp.int32),
        pltpu.VMEM((2, SBLK, BBLK, D), jnp.float32),
        pltpu.VMEM((SBLK, D // 8, 1, 8, BBLK), jnp.float32),
        pltpu.VMEM((S, D), jnp.float32),
        pltpu.SemaphoreType.DMA,
        pltpu.SemaphoreType.DMA,
    ],
    compiler_params=pltpu.CompilerParams(
        use_tc_tiling_on_sc=False, needs_layout_passes=False
    ),
)
def _emb_lookup(xt_hbm, emb_hbm, pos_hbm, out_hbm, idx_v, rows_v, trans_v,
                pos_v, sem0, sem1):
    wid = lax.axis_index("s") * NC + lax.axis_index("c")
    b0 = wid * BBLK
    pltpu.sync_copy(pos_hbm, pos_v)
    lane = lax.iota(jnp.int32, LANES)
    zero16 = jnp.zeros((LANES,), jnp.int32)
    sems = [sem0, sem1]

    def issue(slot, blk, sem):
        pltpu.sync_copy(
            xt_hbm.at[pl.ds(blk, 1), pl.ds(wid, 1)], idx_v.at[slot]
        )
        for i in range(SBLK):
            pltpu.async_copy(
                emb_hbm.at[idx_v.at[slot, 0, 0, i]], rows_v.at[slot, i], sem
            )

    issue(0, 0, sem0)

    def block_body(blk, _):
        slot = blk & 1
        nxt = jnp.minimum(blk + 1, NBLK - 1)
        s0 = blk * SBLK

        for half, sem in enumerate(sems):
            @pl.when(slot != half)
            def _():
                issue(1 - slot, nxt, sem)

        for half, sem in enumerate(sems):
            @pl.when(slot == half)
            def _():
                for i in range(SBLK):
                    pltpu.make_async_copy(
                        emb_hbm.at[idx_v.at[slot, 0, 0, i]],
                        rows_v.at[slot, i],
                        sem,
                    ).wait()

        def seq_body(i, _):
            src = rows_v.at[slot, i]
            dst = trans_v.at[i]
            srow = jnp.broadcast_to(s0 + i, (LANES,))

            @plsc.parallel_loop(0, D, unroll=2)
            def dloop(t):
                tt = jnp.broadcast_to(t, (LANES,))
                dcol = (tt & 16) + ((lane + tt) & 15)
                p = plsc.load_gather(pos_v, [srow, dcol])
                dblk = dcol >> 3
                dsub = dcol & 7
                for jg in range(BBLK // LANES):
                    brow = jg * LANES + lane
                    val = plsc.load_gather(src, [brow, dcol]) + p
                    plsc.store_scatter(dst, [dblk, zero16, dsub, brow], val)

            return 0

        lax.fori_loop(0, SBLK, seq_body, 0)
        pltpu.sync_copy(
            trans_v,
            out_hbm.at[pl.ds(s0, SBLK), :, pl.ds(wid, 1)],
        )
        return 0

    lax.fori_loop(0, NBLK, block_body, 0)
    for i in range(SBLK):
        pltpu.make_async_copy(
            emb_hbm.at[idx_v.at[1, 0, 0, i]], rows_v.at[1, i], sem1
        ).wait()


def kernel(x, embedding, pos_embedding):
    x4 = x.reshape(NW, BBLK, NBLK, SBLK).transpose(2, 0, 3, 1)
    out5 = _emb_lookup(x4, embedding, pos_embedding)
    return out5.transpose(2, 4, 0, 1, 3).reshape(B, S, D)


# speedup vs baseline: 1.0024x; 1.0024x over previous
import functools

import jax
import jax.numpy as jnp
from jax import lax
from jax.experimental import pallas as pl
from jax.experimental.pallas import tpu as pltpu
from jax.experimental.pallas import tpu_sc as plsc

B = 4096
S = 200
D = 32
V = 1000000
NC = 2
NS = 16
NW = NC * NS
BBLK = B // NW
SBLK = 8
NBLK = S // SBLK
LANES = 16


@functools.partial(
    pl.kernel,
    out_type=jax.ShapeDtypeStruct((S, D // 8, NW, 8, BBLK), jnp.float32),
    mesh=plsc.VectorSubcoreMesh(core_axis_name="c", subcore_axis_name="s"),
    scratch_types=[
        pltpu.VMEM((2, 1, 1, SBLK, BBLK), jnp.int32),
        pltpu.VMEM((2, SBLK, BBLK, D), jnp.float32),
        pltpu.VMEM((SBLK, D // 8, 1, 8, BBLK), jnp.float32),
        pltpu.VMEM((S, D), jnp.float32),
        pltpu.SemaphoreType.DMA,
        pltpu.SemaphoreType.DMA,
    ],
    compiler_params=pltpu.CompilerParams(
        use_tc_tiling_on_sc=False, needs_layout_passes=False
    ),
)
def _emb_lookup(xt_hbm, emb_hbm, pos_hbm, out_hbm, idx_v, rows_v, trans_v,
                pos_v, sem0, sem1):
    wid = lax.axis_index("s") * NC + lax.axis_index("c")
    pltpu.sync_copy(pos_hbm, pos_v)
    lane = lax.iota(jnp.int32, LANES)
    zero16 = jnp.zeros((LANES,), jnp.int32)
    sems = [sem0, sem1]

    def issue(slot, blk, sem):
        pltpu.sync_copy(
            xt_hbm.at[pl.ds(blk, 1), pl.ds(wid, 1)], idx_v.at[slot]
        )
        for i in range(SBLK):
            pltpu.async_copy(
                emb_hbm.at[idx_v.at[slot, 0, 0, i]], rows_v.at[slot, i], sem
            )

    issue(0, 0, sem0)

    def block_body(blk, _):
        slot = blk & 1
        nxt = jnp.minimum(blk + 1, NBLK - 1)
        s0 = blk * SBLK

        for half, sem in enumerate(sems):
            @pl.when(slot != half)
            def _():
                issue(1 - slot, nxt, sem)

        for half, sem in enumerate(sems):
            @pl.when(slot == half)
            def _():
                for i in range(SBLK):
                    pltpu.make_async_copy(
                        emb_hbm.at[idx_v.at[slot, 0, 0, i]],
                        rows_v.at[slot, i],
                        sem,
                    ).wait()

        def seq_body(i, _):
            src = rows_v.at[slot, i]
            dst = trans_v.at[i]
            srow = jnp.broadcast_to(s0 + i, (LANES,))

            @plsc.parallel_loop(0, D, unroll=2)
            def dloop(t):
                tt = jnp.broadcast_to(t, (LANES,))
                dcol = (tt & 16) + ((lane + tt) & 15)
                p = plsc.load_gather(pos_v, [srow, dcol])
                dblk = dcol >> 3
                dsub = dcol & 7
                for jg in range(BBLK // LANES):
                    brow = jg * LANES + lane
                    val = plsc.load_gather(src, [brow, dcol]) + p
                    plsc.store_scatter(dst, [dblk, zero16, dsub, brow], val)

            return 0

        lax.fori_loop(0, SBLK, seq_body, 0)
        pltpu.sync_copy(
            trans_v,
            out_hbm.at[pl.ds(s0, SBLK), :, pl.ds(wid, 1)],
        )
        return 0

    lax.fori_loop(0, NBLK, block_body, 0)
    for i in range(SBLK):
        pltpu.make_async_copy(
            emb_hbm.at[idx_v.at[1, 0, 0, i]], rows_v.at[1, i], sem1
        ).wait()


def kernel(x, embedding, pos_embedding):
    x4 = x.reshape(NW, BBLK, NBLK, SBLK).transpose(2, 0, 3, 1)
    out5 = _emb_lookup(x4, embedding, pos_embedding)
    return out5.transpose(2, 4, 0, 1, 3).reshape(B, S, D)
